# initial kernel scaffold (unmeasured)
import jax
import jax.numpy as jnp
from jax import lax
from jax.experimental import pallas as pl
from jax.experimental.pallas import tpu as pltpu

B, S, D = 1, 1024, 2048
H, Dh, Dr = 16, 128, 32
DC_HALF = 128


def kernel(x, Wdkv, Wuk, Wuv, Wq, Wqr, Wkr, Wo):
    def body(
        x_ref, wdkv_ref, wuk_ref, wuv_ref, wq_ref, wqr_ref, wkr_ref, wo_ref,
        out_ref,
        c_buf, c_rem, wuk_rem, wuv_rem, send_sems, recv_sems,
    ):
        my_x = lax.axis_index("x")
        my_y = lax.axis_index("y")
        nbr = (1 - my_x, my_y)

        barrier_sem = pltpu.get_barrier_semaphore()
        pl.semaphore_signal(
            barrier_sem, inc=1, device_id=nbr,
            device_id_type=pl.DeviceIdType.MESH,
        )
        pl.semaphore_wait(barrier_sem, 1)

        x2d = x_ref[0]

        c_loc = jnp.dot(x2d, wdkv_ref[...], preferred_element_type=jnp.float32)
        c_buf[...] = c_loc

        rdma_c = pltpu.make_async_remote_copy(
            src_ref=c_buf, dst_ref=c_rem,
            send_sem=send_sems.at[0], recv_sem=recv_sems.at[0],
            device_id=nbr, device_id_type=pl.DeviceIdType.MESH,
        )
        rdma_k = pltpu.make_async_remote_copy(
            src_ref=wuk_ref, dst_ref=wuk_rem,
            send_sem=send_sems.at[1], recv_sem=recv_sems.at[1],
            device_id=nbr, device_id_type=pl.DeviceIdType.MESH,
        )
        rdma_v = pltpu.make_async_remote_copy(
            src_ref=wuv_ref, dst_ref=wuv_rem,
            send_sem=send_sems.at[2], recv_sem=recv_sems.at[2],
            device_id=nbr, device_id_type=pl.DeviceIdType.MESH,
        )
        rdma_c.start()
        rdma_k.start()
        rdma_v.start()

        Q = jnp.dot(x2d, wq_ref[...], preferred_element_type=jnp.float32)
        Qr = jnp.dot(x2d, wqr_ref[...], preferred_element_type=jnp.float32)
        Kr = jnp.dot(x2d, wkr_ref[...], preferred_element_type=jnp.float32)

        rdma_c.wait()
        rdma_k.wait()
        rdma_v.wait()

        K = (
            jnp.dot(c_loc, wuk_ref[...], preferred_element_type=jnp.float32)
            + jnp.dot(c_rem[...], wuk_rem[...], preferred_element_type=jnp.float32)
        )
        V = (
            jnp.dot(c_loc, wuv_ref[...], preferred_element_type=jnp.float32)
            + jnp.dot(c_rem[...], wuv_rem[...], preferred_element_type=jnp.float32)
        )

        scale = (Dh + Dr) ** -0.5
        dn = (((1,), (1,)), ((), ()))
        acc = jnp.zeros((S, D), dtype=jnp.float32)
        for h in range(H):
            Qh = Q[:, h * Dh:(h + 1) * Dh]
            Kh = K[:, h * Dh:(h + 1) * Dh]
            Vh = V[:, h * Dh:(h + 1) * Dh]
            Qrh = Qr[:, h * Dr:(h + 1) * Dr]
            s = (
                lax.dot_general(Qh, Kh, dn, preferred_element_type=jnp.float32)
                + lax.dot_general(Qrh, Kr, dn, preferred_element_type=jnp.float32)
            ) * scale
            m = jnp.max(s, axis=-1, keepdims=True)
            p = jnp.exp(s - m)
            p = p / jnp.sum(p, axis=-1, keepdims=True)
            Oh = jnp.dot(p, Vh, preferred_element_type=jnp.float32)
            acc = acc + jnp.dot(
                Oh, wo_ref[h * Dh:(h + 1) * Dh, :],
                preferred_element_type=jnp.float32,
            )
        out_ref[0] = acc

    return pl.pallas_call(
        body,
        out_shape=jax.ShapeDtypeStruct((B, S, D), jnp.float32),
        in_specs=[pl.BlockSpec(memory_space=pltpu.VMEM)] * 8,
        out_specs=pl.BlockSpec(memory_space=pltpu.VMEM),
        scratch_shapes=[
            pltpu.VMEM((S, DC_HALF), jnp.float32),
            pltpu.VMEM((S, DC_HALF), jnp.float32),
            pltpu.VMEM((DC_HALF, D), jnp.float32),
            pltpu.VMEM((DC_HALF, D), jnp.float32),
            pltpu.SemaphoreType.DMA((3,)),
            pltpu.SemaphoreType.DMA((3,)),
        ],
        compiler_params=pltpu.CompilerParams(collective_id=0),
    )(x, Wdkv, Wuk, Wuv, Wq, Wqr, Wkr, Wo)


# baseline (device time: 168959 ns/iter reference)
import jax
import jax.numpy as jnp
from jax import lax
from jax.experimental import pallas as pl
from jax.experimental.pallas import tpu as pltpu

B, S, D = 1, 1024, 2048
H, Dh, Dr = 16, 128, 32
DC_HALF = 128


def kernel(x, Wdkv, Wuk, Wuv, Wq, Wqr, Wkr, Wo):
    WukT = Wuk.reshape(DC_HALF, H, Dh).transpose(1, 0, 2)
    WuvT = Wuv.reshape(DC_HALF, H, Dh).transpose(1, 0, 2)
    WqrT = Wqr.reshape(D, H, Dr).transpose(1, 0, 2)

    def body(
        x_ref, wdkv_ref, wuk_ref, wuv_ref, wq_ref, wqr_ref, wkr_ref, wo_ref,
        out_ref,
        c_buf, c_rem, wuk_rem, wuv_rem, kr_buf, send_sems, recv_sems,
    ):
        h = pl.program_id(0)
        my_x = lax.axis_index("x")
        my_y = lax.axis_index("y")
        nbr = (1 - my_x, my_y)

        @pl.when(h == 0)
        def _exchange():
            barrier_sem = pltpu.get_barrier_semaphore()
            pl.semaphore_signal(
                barrier_sem, inc=1, device_id=nbr,
                device_id_type=pl.DeviceIdType.MESH,
            )
            pl.semaphore_wait(barrier_sem, 1)

            x2d = x_ref[0]
            c_buf[...] = jnp.dot(
                x2d, wdkv_ref[...], preferred_element_type=jnp.float32
            )
            kr_buf[...] = jnp.dot(
                x2d, wkr_ref[...], preferred_element_type=jnp.float32
            )

            rdma_c = pltpu.make_async_remote_copy(
                src_ref=c_buf, dst_ref=c_rem,
                send_sem=send_sems.at[0], recv_sem=recv_sems.at[0],
                device_id=nbr, device_id_type=pl.DeviceIdType.MESH,
            )
            rdma_k = pltpu.make_async_remote_copy(
                src_ref=wuk_ref, dst_ref=wuk_rem,
                send_sem=send_sems.at[1], recv_sem=recv_sems.at[1],
                device_id=nbr, device_id_type=pl.DeviceIdType.MESH,
            )
            rdma_v = pltpu.make_async_remote_copy(
                src_ref=wuv_ref, dst_ref=wuv_rem,
                send_sem=send_sems.at[2], recv_sem=recv_sems.at[2],
                device_id=nbr, device_id_type=pl.DeviceIdType.MESH,
            )
            rdma_c.start()
            rdma_k.start()
            rdma_v.start()
            rdma_c.wait()
            rdma_k.wait()
            rdma_v.wait()

            out_ref[0] = jnp.zeros((S, D), dtype=jnp.float32)

        x2d = x_ref[0]
        c_loc = c_buf[...]
        c_rm = c_rem[...]

        Kh = (
            jnp.dot(c_loc, wuk_ref[h], preferred_element_type=jnp.float32)
            + jnp.dot(c_rm, wuk_rem[h], preferred_element_type=jnp.float32)
        )
        Vh = (
            jnp.dot(c_loc, wuv_ref[h], preferred_element_type=jnp.float32)
            + jnp.dot(c_rm, wuv_rem[h], preferred_element_type=jnp.float32)
        )
        Qh = jnp.dot(x2d, wq_ref[...], preferred_element_type=jnp.float32)
        Qrh = jnp.dot(x2d, wqr_ref[0], preferred_element_type=jnp.float32)

        scale = (Dh + Dr) ** -0.5
        dn = (((1,), (1,)), ((), ()))
        s = (
            lax.dot_general(Qh, Kh, dn, preferred_element_type=jnp.float32)
            + lax.dot_general(
                Qrh, kr_buf[...], dn, preferred_element_type=jnp.float32
            )
        ) * scale
        m = jnp.max(s, axis=-1, keepdims=True)
        p = jnp.exp(s - m)
        p = p / jnp.sum(p, axis=-1, keepdims=True)
        Oh = jnp.dot(p, Vh, preferred_element_type=jnp.float32)
        out_ref[0] += jnp.dot(
            Oh, wo_ref[...], preferred_element_type=jnp.float32
        )

    return pl.pallas_call(
        body,
        grid=(H,),
        out_shape=jax.ShapeDtypeStruct((B, S, D), jnp.float32),
        in_specs=[
            pl.BlockSpec((1, S, D), lambda h: (0, 0, 0)),
            pl.BlockSpec((D, DC_HALF), lambda h: (0, 0)),
            pl.BlockSpec((H, DC_HALF, Dh), lambda h: (0, 0, 0)),
            pl.BlockSpec((H, DC_HALF, Dh), lambda h: (0, 0, 0)),
            pl.BlockSpec((D, Dh), lambda h: (0, h)),
            pl.BlockSpec((1, D, Dr), lambda h: (h, 0, 0)),
            pl.BlockSpec((D, Dr), lambda h: (0, 0)),
            pl.BlockSpec((Dh, D), lambda h: (h, 0)),
        ],
        out_specs=pl.BlockSpec((1, S, D), lambda h: (0, 0, 0)),
        scratch_shapes=[
            pltpu.VMEM((S, DC_HALF), jnp.float32),
            pltpu.VMEM((S, DC_HALF), jnp.float32),
            pltpu.VMEM((H, DC_HALF, Dh), jnp.float32),
            pltpu.VMEM((H, DC_HALF, Dh), jnp.float32),
            pltpu.VMEM((S, Dr), jnp.float32),
            pltpu.SemaphoreType.DMA((3,)),
            pltpu.SemaphoreType.DMA((3,)),
        ],
        compiler_params=pltpu.CompilerParams(collective_id=0),
    )(x, Wdkv, WukT, WuvT, Wq, WqrT, Wkr, Wo)


# device time: 148582 ns/iter; 1.1371x vs baseline; 1.1371x over previous
import jax
import jax.numpy as jnp
from jax import lax
from jax.experimental import pallas as pl
from jax.experimental.pallas import tpu as pltpu

B, S, D = 1, 1024, 2048
H, Dh, Dr = 16, 128, 32
DC_HALF = 128
DC = 256
NROW = 4
SR = S // NROW


def kernel(x, Wdkv, Wuk, Wuv, Wq, Wqr, Wkr, Wo):
    WukT = Wuk.reshape(DC_HALF, H, Dh).transpose(1, 0, 2)
    WuvT = Wuv.reshape(DC_HALF, H, Dh).transpose(1, 0, 2)
    WqrT = Wqr.reshape(D, H, Dr).transpose(1, 0, 2)

    def body(
        x_ref, wdkv_ref, wuk_ref, wuv_ref, wq_ref, wqr_ref, wkr_ref, wo_ref,
        out_ref,
        c_buf, c_cat, wuk_cat, wuv_cat, kr_buf,
        ex_send_sems, ex_recv_sems, ag_send_sems, ag_recv_sems,
    ):
        h = pl.program_id(0)
        my_x = lax.axis_index("x")
        my_y = lax.axis_index("y")
        rid = 2 * my_x + my_y
        x_nbr = (1 - my_x, my_y)
        y_nbr = (my_x, 1 - my_y)
        diag = (1 - my_x, 1 - my_y)
        peers = (x_nbr, y_nbr, diag)

        @pl.when(h == 0)
        def _exchange():
            barrier_sem = pltpu.get_barrier_semaphore()
            for p in peers:
                pl.semaphore_signal(
                    barrier_sem, inc=1, device_id=p,
                    device_id_type=pl.DeviceIdType.MESH,
                )
            pl.semaphore_wait(barrier_sem, 3)

            x2d = x_ref[0]
            c_loc = jnp.dot(
                x2d, wdkv_ref[...], preferred_element_type=jnp.float32
            )
            c_buf[...] = c_loc
            kr_buf[...] = jnp.dot(
                x2d, wkr_ref[...], preferred_element_type=jnp.float32
            )

            rdma_c = pltpu.make_async_remote_copy(
                src_ref=c_buf, dst_ref=c_cat.at[:, DC_HALF:],
                send_sem=ex_send_sems.at[0], recv_sem=ex_recv_sems.at[0],
                device_id=x_nbr, device_id_type=pl.DeviceIdType.MESH,
            )
            rdma_k = pltpu.make_async_remote_copy(
                src_ref=wuk_ref, dst_ref=wuk_cat.at[:, DC_HALF:, :],
                send_sem=ex_send_sems.at[1], recv_sem=ex_recv_sems.at[1],
                device_id=x_nbr, device_id_type=pl.DeviceIdType.MESH,
            )
            rdma_v = pltpu.make_async_remote_copy(
                src_ref=wuv_ref, dst_ref=wuv_cat.at[:, DC_HALF:, :],
                send_sem=ex_send_sems.at[2], recv_sem=ex_recv_sems.at[2],
                device_id=x_nbr, device_id_type=pl.DeviceIdType.MESH,
            )
            rdma_c.start()
            rdma_k.start()
            rdma_v.start()

            c_cat[:, :DC_HALF] = c_loc
            wuk_cat[:, :DC_HALF, :] = wuk_ref[...]
            wuv_cat[:, :DC_HALF, :] = wuv_ref[...]

            rdma_c.wait()
            rdma_k.wait()
            rdma_v.wait()

        x_mine = x_ref[0, pl.ds(rid * SR, SR), :]
        c_full = c_cat[...]

        Kh = jnp.dot(c_full, wuk_cat[h], preferred_element_type=jnp.float32)
        Vh = jnp.dot(c_full, wuv_cat[h], preferred_element_type=jnp.float32)
        Qh = jnp.dot(x_mine, wq_ref[...], preferred_element_type=jnp.float32)
        Qrh = jnp.dot(x_mine, wqr_ref[0], preferred_element_type=jnp.float32)

        scale = (Dh + Dr) ** -0.5
        dn = (((1,), (1,)), ((), ()))
        s = (
            lax.dot_general(Qh, Kh, dn, preferred_element_type=jnp.float32)
            + lax.dot_general(
                Qrh, kr_buf[...], dn, preferred_element_type=jnp.float32
            )
        ) * scale
        m = jnp.max(s, axis=-1, keepdims=True)
        p = jnp.exp(s - m)
        p = p / jnp.sum(p, axis=-1, keepdims=True)
        Oh = jnp.dot(p, Vh, preferred_element_type=jnp.float32)
        contrib = jnp.dot(Oh, wo_ref[...], preferred_element_type=jnp.float32)

        @pl.when(h == 0)
        def _init():
            out_ref[rid] = contrib

        @pl.when(h > 0)
        def _acc():
            out_ref[rid] += contrib

        @pl.when(h == H - 1)
        def _allgather():
            rdmas = []
            for i, p in enumerate(peers):
                r = pltpu.make_async_remote_copy(
                    src_ref=out_ref.at[rid],
                    dst_ref=out_ref.at[rid],
                    send_sem=ag_send_sems.at[i], recv_sem=ag_recv_sems.at[i],
                    device_id=p, device_id_type=pl.DeviceIdType.MESH,
                )
                r.start()
                rdmas.append(r)
            for r in rdmas:
                r.wait()

    out4 = pl.pallas_call(
        body,
        grid=(H,),
        out_shape=jax.ShapeDtypeStruct((NROW, SR, D), jnp.float32),
        in_specs=[
            pl.BlockSpec((1, S, D), lambda h: (0, 0, 0)),
            pl.BlockSpec((D, DC_HALF), lambda h: (0, 0)),
            pl.BlockSpec((H, DC_HALF, Dh), lambda h: (0, 0, 0)),
            pl.BlockSpec((H, DC_HALF, Dh), lambda h: (0, 0, 0)),
            pl.BlockSpec((D, Dh), lambda h: (0, h)),
            pl.BlockSpec((1, D, Dr), lambda h: (h, 0, 0)),
            pl.BlockSpec((D, Dr), lambda h: (0, 0)),
            pl.BlockSpec((Dh, D), lambda h: (h, 0)),
        ],
        out_specs=pl.BlockSpec((NROW, SR, D), lambda h: (0, 0, 0)),
        scratch_shapes=[
            pltpu.VMEM((S, DC_HALF), jnp.float32),
            pltpu.VMEM((S, DC), jnp.float32),
            pltpu.VMEM((H, DC, Dh), jnp.float32),
            pltpu.VMEM((H, DC, Dh), jnp.float32),
            pltpu.VMEM((S, Dr), jnp.float32),
            pltpu.SemaphoreType.DMA((3,)),
            pltpu.SemaphoreType.DMA((3,)),
            pltpu.SemaphoreType.DMA((3,)),
            pltpu.SemaphoreType.DMA((3,)),
        ],
        compiler_params=pltpu.CompilerParams(collective_id=0),
    )(x, Wdkv, WukT, WuvT, Wq, WqrT, Wkr, Wo)
    return out4.reshape(B, S, D)


# device time: 146605 ns/iter; 1.1525x vs baseline; 1.0135x over previous
import jax
import jax.numpy as jnp
from jax import lax
from jax.experimental import pallas as pl
from jax.experimental.pallas import tpu as pltpu

B, S, D = 1, 1024, 2048
H, Dh, Dr = 16, 128, 32
DC_HALF = 128
DC = 256
NROW = 4
SR = S // NROW
NCK = 2
CH = SR // NCK


def kernel(x, Wdkv, Wuk, Wuv, Wq, Wqr, Wkr, Wo):
    WukT = Wuk.reshape(DC_HALF, H, Dh).transpose(1, 0, 2)
    WuvT = Wuv.reshape(DC_HALF, H, Dh).transpose(1, 0, 2)
    WqrT = Wqr.reshape(D, H, Dr).transpose(1, 0, 2)

    def body(
        x_ref, wdkv_ref, wuk_ref, wuv_ref, wq_ref, wqr_ref, wkr_ref, wo_ref,
        out_ref,
        c_buf, c_cat, wuk_cat, wuv_cat, k_buf, kr_buf,
        c_send_sem, c_recv_sem, wk_send_sems, wk_recv_sems,
        wv_send_sems, wv_recv_sems, ag_send_sems, ag_recv_sems,
    ):
        ck = pl.program_id(0)
        h = pl.program_id(1)
        my_x = lax.axis_index("x")
        my_y = lax.axis_index("y")
        rid = 2 * my_x + my_y
        x_nbr = (1 - my_x, my_y)
        y_nbr = (my_x, 1 - my_y)
        diag = (1 - my_x, 1 - my_y)
        peers = (x_nbr, y_nbr, diag)

        def wuk_rdma(hh):
            return pltpu.make_async_remote_copy(
                src_ref=wuk_ref.at[hh],
                dst_ref=wuk_cat.at[hh, DC_HALF:, :],
                send_sem=wk_send_sems.at[hh], recv_sem=wk_recv_sems.at[hh],
                device_id=x_nbr, device_id_type=pl.DeviceIdType.MESH,
            )

        def wuv_rdma(hh):
            return pltpu.make_async_remote_copy(
                src_ref=wuv_ref.at[hh],
                dst_ref=wuv_cat.at[hh, DC_HALF:, :],
                send_sem=wv_send_sems.at[hh], recv_sem=wv_recv_sems.at[hh],
                device_id=x_nbr, device_id_type=pl.DeviceIdType.MESH,
            )

        def c_rdma():
            return pltpu.make_async_remote_copy(
                src_ref=c_buf, dst_ref=c_cat.at[:, DC_HALF:],
                send_sem=c_send_sem, recv_sem=c_recv_sem,
                device_id=x_nbr, device_id_type=pl.DeviceIdType.MESH,
            )

        def ag_rdma(cck, i, p):
            return pltpu.make_async_remote_copy(
                src_ref=out_ref.at[rid, pl.ds(cck * CH, CH), :],
                dst_ref=out_ref.at[rid, pl.ds(cck * CH, CH), :],
                send_sem=ag_send_sems.at[cck, i],
                recv_sem=ag_recv_sems.at[cck, i],
                device_id=p, device_id_type=pl.DeviceIdType.MESH,
            )

        @pl.when(jnp.logical_and(ck == 0, h == 0))
        def _exchange():
            barrier_sem = pltpu.get_barrier_semaphore()
            for p in peers:
                pl.semaphore_signal(
                    barrier_sem, inc=1, device_id=p,
                    device_id_type=pl.DeviceIdType.MESH,
                )
            pl.semaphore_wait(barrier_sem, 3)

            for hh in range(H):
                wuk_rdma(hh).start()
                wuv_rdma(hh).start()

            x2d = x_ref[0]
            c_loc = jnp.dot(
                x2d, wdkv_ref[...], preferred_element_type=jnp.float32
            )
            c_buf[...] = c_loc
            c_rdma().start()
            kr_buf[...] = jnp.dot(
                x2d, wkr_ref[...], preferred_element_type=jnp.float32
            )

            c_cat[:, :DC_HALF] = c_loc
            wuk_cat[:, :DC_HALF, :] = wuk_ref[...]
            wuv_cat[:, :DC_HALF, :] = wuv_ref[...]

            c_rdma().wait_recv()

        @pl.when(ck == 0)
        def _build_kv():
            wuk_rdma(h).wait_recv()
            wuv_rdma(h).wait_recv()
            k_buf[h] = jnp.dot(
                c_cat[...], wuk_cat[h], preferred_element_type=jnp.float32
            )

        Kh = k_buf[h]
        Vh = jnp.dot(
            c_cat[...], wuv_cat[h], preferred_element_type=jnp.float32
        )
        x_mine = x_ref[0, pl.ds(rid * SR + ck * CH, CH), :]
        Qh = jnp.dot(x_mine, wq_ref[...], preferred_element_type=jnp.float32)
        Qrh = jnp.dot(x_mine, wqr_ref[0], preferred_element_type=jnp.float32)

        scale = (Dh + Dr) ** -0.5
        dn = (((1,), (1,)), ((), ()))
        s = (
            lax.dot_general(Qh, Kh, dn, preferred_element_type=jnp.float32)
            + lax.dot_general(
                Qrh, kr_buf[...], dn, preferred_element_type=jnp.float32
            )
        ) * scale
        m = jnp.max(s, axis=-1, keepdims=True)
        p = jnp.exp(s - m)
        p = p / jnp.sum(p, axis=-1, keepdims=True)
        Oh = jnp.dot(p, Vh, preferred_element_type=jnp.float32)
        contrib = jnp.dot(Oh, wo_ref[...], preferred_element_type=jnp.float32)

        row = pl.ds(ck * CH, CH)

        @pl.when(h == 0)
        def _init():
            out_ref[rid, row, :] = contrib

        @pl.when(h > 0)
        def _acc():
            out_ref[rid, row, :] += contrib

        @pl.when(jnp.logical_and(ck == 0, h == H - 1))
        def _push_chunk0():
            for i, p in enumerate(peers):
                ag_rdma(0, i, p).start()

        @pl.when(jnp.logical_and(ck == 1, h == H - 1))
        def _push_chunk1_and_drain():
            for i, p in enumerate(peers):
                ag_rdma(1, i, p).start()
            for cck in range(NCK):
                for i, p in enumerate(peers):
                    r = ag_rdma(cck, i, p)
                    r.wait_send()
                    r.wait_recv()
            c_rdma().wait_send()
            for hh in range(H):
                wuk_rdma(hh).wait_send()
                wuv_rdma(hh).wait_send()

    out4 = pl.pallas_call(
        body,
        grid=(NCK, H),
        out_shape=jax.ShapeDtypeStruct((NROW, SR, D), jnp.float32),
        in_specs=[
            pl.BlockSpec((1, S, D), lambda ck, h: (0, 0, 0)),
            pl.BlockSpec((D, DC_HALF), lambda ck, h: (0, 0)),
            pl.BlockSpec((H, DC_HALF, Dh), lambda ck, h: (0, 0, 0)),
            pl.BlockSpec((H, DC_HALF, Dh), lambda ck, h: (0, 0, 0)),
            pl.BlockSpec((D, Dh), lambda ck, h: (0, h)),
            pl.BlockSpec((1, D, Dr), lambda ck, h: (h, 0, 0)),
            pl.BlockSpec((D, Dr), lambda ck, h: (0, 0)),
            pl.BlockSpec((Dh, D), lambda ck, h: (h, 0)),
        ],
        out_specs=pl.BlockSpec((NROW, SR, D), lambda ck, h: (0, 0, 0)),
        scratch_shapes=[
            pltpu.VMEM((S, DC_HALF), jnp.float32),
            pltpu.VMEM((S, DC), jnp.float32),
            pltpu.VMEM((H, DC, Dh), jnp.float32),
            pltpu.VMEM((H, DC, Dh), jnp.float32),
            pltpu.VMEM((H, S, Dh), jnp.float32),
            pltpu.VMEM((S, Dr), jnp.float32),
            pltpu.SemaphoreType.DMA,
            pltpu.SemaphoreType.DMA,
            pltpu.SemaphoreType.DMA((H,)),
            pltpu.SemaphoreType.DMA((H,)),
            pltpu.SemaphoreType.DMA((H,)),
            pltpu.SemaphoreType.DMA((H,)),
            pltpu.SemaphoreType.DMA((NCK, 3)),
            pltpu.SemaphoreType.DMA((NCK, 3)),
        ],
        compiler_params=pltpu.CompilerParams(
            collective_id=0, vmem_limit_bytes=38 * 1024 * 1024
        ),
    )(x, Wdkv, WukT, WuvT, Wq, WqrT, Wkr, Wo)
    return out4.reshape(B, S, D)
